# baseline (speedup 1.0000x reference)
import functools

import jax
import jax.numpy as jnp
from jax import lax
from jax.experimental import pallas as pl
from jax.experimental.pallas import tpu as pltpu
import jax.experimental.pallas.tpu_sc as plsc

N_NODES = 10000
N_EDGES = 320000
D = 128
L = 16
NC, NS = 2, 16
NW = NC * NS
EPW = N_EDGES // NW
B = 80
NB = EPW // B

_MESH = dict(core_axis_name="c", subcore_axis_name="s", num_cores=NC,
             num_subcores=NS)


def _scores_body(emb_ref, w2_ref, out_ref):
    out_ref[...] = lax.dot_general(
        w2_ref[...], emb_ref[...], (((1,), (1,)), ((), ())),
        preferred_element_type=jnp.float32)


def _node_scores(emb, w2):
    return pl.pallas_call(
        _scores_body,
        out_shape=jax.ShapeDtypeStruct((2, N_NODES), jnp.float32),
    )(emb, w2)


def _logits_body(s_hbm, t_hbm, src_hbm, dst_hbm, out_hbm,
                 s_v, t_v, src_v, dst_v, lg_v):
    wid = lax.axis_index("s") * NC + lax.axis_index("c")
    base = wid * EPW
    pltpu.sync_copy(s_hbm, s_v)
    pltpu.sync_copy(t_hbm, t_v)
    pltpu.sync_copy(src_hbm.at[pl.ds(base, EPW)], src_v)
    pltpu.sync_copy(dst_hbm.at[pl.ds(base, EPW)], dst_v)

    def body(i, carry):
        o = i * L
        is_ = src_v[pl.ds(o, L)]
        id_ = dst_v[pl.ds(o, L)]
        sv = plsc.load_gather(s_v, [is_])
        tv = plsc.load_gather(t_v, [id_])
        lg_v[pl.ds(o, L)] = sv + tv
        return carry

    lax.fori_loop(0, EPW // L, body, 0)
    pltpu.sync_copy(lg_v, out_hbm.at[pl.ds(base, EPW)])


def _edge_logits(s, t, src, dst):
    k = functools.partial(
        pl.kernel,
        out_type=jax.ShapeDtypeStruct((N_EDGES,), jnp.float32),
        mesh=plsc.VectorSubcoreMesh(**_MESH),
        compiler_params=pltpu.CompilerParams(needs_layout_passes=False),
        scratch_types=[
            pltpu.VMEM((N_NODES,), jnp.float32),
            pltpu.VMEM((N_NODES,), jnp.float32),
            pltpu.VMEM((EPW,), jnp.int32),
            pltpu.VMEM((EPW,), jnp.int32),
            pltpu.VMEM((EPW,), jnp.float32),
        ],
    )(_logits_body)
    return k(s, t, src, dst)


def _softmax_body(x_ref, o_ref):
    x = x_ref[...]
    m = jnp.max(x)
    e = jnp.exp(x - m)
    o_ref[...] = e / jnp.sum(e)


def _softmax(logits2d):
    return pl.pallas_call(
        _softmax_body,
        out_shape=jax.ShapeDtypeStruct(logits2d.shape, jnp.float32),
    )(logits2d)


def _scale_body(emb_hbm, src_hbm, dst_hbm, w_hbm, out_hbm,
                si, di, wq, rs, rd, sh_emb, gsem, osem, isem):
    wid = lax.axis_index("s") * NC + lax.axis_index("c")
    base = wid * EPW
    @pl.when(lax.axis_index("s") == 0)
    def _():
        pltpu.sync_copy(emb_hbm, sh_emb)

    plsc.subcore_barrier()

    def i_copies(k, s):
        return (
            pltpu.make_async_copy(
                src_hbm.at[pl.ds(base + k * B, B)], si.at[s], isem.at[s]),
            pltpu.make_async_copy(
                dst_hbm.at[pl.ds(base + k * B, B)], di.at[s], isem.at[s]),
            pltpu.make_async_copy(
                w_hbm.at[pl.ds(base + k * B, B)], wq.at[s], isem.at[s]),
        )

    def g_copies(k, s):
        return (
            pltpu.make_async_copy(
                emb_hbm.at[si.at[s]], rs.at[s], gsem.at[s]),
            pltpu.make_async_copy(
                emb_hbm.at[di.at[s]], rd.at[s], gsem.at[s]),
        )

    def o_copies(k, s):
        return (
            pltpu.make_async_copy(
                rs.at[s],
                out_hbm.at[pl.ds(base + k * B, B), pl.ds(0, D)],
                osem.at[s]),
            pltpu.make_async_copy(
                rd.at[s],
                out_hbm.at[pl.ds(base + k * B, B), pl.ds(D, D)],
                osem.at[s]),
        )

    def start(copies):
        for c in copies:
            c.start()

    def wait(copies):
        for c in copies:
            c.wait()

    def compute(s):
        @plsc.parallel_loop(0, B, step=1, unroll=4)
        def _(e):
            wb = plsc.load_gather(wq.at[s], [jnp.full((L,), e, jnp.int32)])
            for f in range(D // L):
                rs[s, e, pl.ds(f * L, L)] = rs[s, e, pl.ds(f * L, L)] * wb
                rd[s, e, pl.ds(f * L, L)] = rd[s, e, pl.ds(f * L, L)] * wb

    start(i_copies(0, 0))
    start(i_copies(1, 1))
    wait(i_copies(0, 0))
    start(g_copies(0, 0))

    def body(k, carry):
        s = lax.rem(k, 2)
        @pl.when(k + 1 < NB)
        def _():
            wait(i_copies(k + 1, 1 - s))
            start(g_copies(k + 1, 1 - s))

        wait(g_copies(k, s))
        compute(s)

        @pl.when(k + 2 < NB)
        def _():
            start(i_copies(k + 2, s))

        start(o_copies(k, s))
        wait(o_copies(k, s))
        return carry

    lax.fori_loop(0, NB, body, 0)


def _gather_scale(emb, src, dst, w):
    k = functools.partial(
        pl.kernel,
        out_type=jax.ShapeDtypeStruct((N_EDGES, 2 * D), jnp.float32),
        mesh=plsc.VectorSubcoreMesh(**_MESH),
        compiler_params=pltpu.CompilerParams(needs_layout_passes=False),
        scratch_types=[
            pltpu.VMEM((2, B), jnp.int32),
            pltpu.VMEM((2, B), jnp.int32),
            pltpu.VMEM((2, B), jnp.float32),
            pltpu.VMEM((2, B, D), jnp.float32),
            pltpu.VMEM((2, B, D), jnp.float32),
            pltpu.VMEM_SHARED((N_NODES, D), jnp.float32),
            pltpu.SemaphoreType.DMA((2,)),
            pltpu.SemaphoreType.DMA((2,)),
            pltpu.SemaphoreType.DMA((2,)),
        ],
    )(_scale_body)
    return k(emb, src, dst, w)


def kernel(node_embeddings, edge_index, attn_w, attn_b):
    emb = node_embeddings.astype(jnp.float32)
    src = edge_index[0].astype(jnp.int32)
    dst = edge_index[1].astype(jnp.int32)
    w2 = attn_w.reshape(2, D)
    st = _node_scores(emb, w2)
    logits = _edge_logits(st[0], st[1], src, dst)
    w = _softmax(logits.reshape(N_EDGES // D, D)).reshape(N_EDGES)
    return _gather_scale(emb, src, dst, w)


# speedup vs baseline: 1.1590x; 1.1590x over previous
import functools

import jax
import jax.numpy as jnp
from jax import lax
from jax.experimental import pallas as pl
from jax.experimental.pallas import tpu as pltpu
import jax.experimental.pallas.tpu_sc as plsc

N_NODES = 10000
N_EDGES = 320000
D = 128
L = 16
NC, NS = 2, 16
NW = NC * NS
EPW = N_EDGES // NW
B = 80
NB = EPW // B

_MESH = dict(core_axis_name="c", subcore_axis_name="s", num_cores=NC,
             num_subcores=NS)


def _scores_body(emb_ref, w2_ref, out_ref):
    out_ref[...] = lax.dot_general(
        w2_ref[...], emb_ref[...], (((1,), (1,)), ((), ())),
        preferred_element_type=jnp.float32)


def _node_scores(emb, w2):
    return pl.pallas_call(
        _scores_body,
        out_shape=jax.ShapeDtypeStruct((2, N_NODES), jnp.float32),
    )(emb, w2)


def _logits_body(s_hbm, t_hbm, src_hbm, dst_hbm, out_hbm,
                 s_v, t_v, src_v, dst_v, lg_v):
    wid = lax.axis_index("s") * NC + lax.axis_index("c")
    base = wid * EPW
    pltpu.sync_copy(s_hbm, s_v)
    pltpu.sync_copy(t_hbm, t_v)
    pltpu.sync_copy(src_hbm.at[pl.ds(base, EPW)], src_v)
    pltpu.sync_copy(dst_hbm.at[pl.ds(base, EPW)], dst_v)

    def body(i, carry):
        o = i * L
        is_ = src_v[pl.ds(o, L)]
        id_ = dst_v[pl.ds(o, L)]
        sv = plsc.load_gather(s_v, [is_])
        tv = plsc.load_gather(t_v, [id_])
        lg_v[pl.ds(o, L)] = sv + tv
        return carry

    lax.fori_loop(0, EPW // L, body, 0)
    pltpu.sync_copy(lg_v, out_hbm.at[pl.ds(base, EPW)])


def _edge_logits(s, t, src, dst):
    k = functools.partial(
        pl.kernel,
        out_type=jax.ShapeDtypeStruct((N_EDGES,), jnp.float32),
        mesh=plsc.VectorSubcoreMesh(**_MESH),
        compiler_params=pltpu.CompilerParams(needs_layout_passes=False),
        scratch_types=[
            pltpu.VMEM((N_NODES,), jnp.float32),
            pltpu.VMEM((N_NODES,), jnp.float32),
            pltpu.VMEM((EPW,), jnp.int32),
            pltpu.VMEM((EPW,), jnp.int32),
            pltpu.VMEM((EPW,), jnp.float32),
        ],
    )(_logits_body)
    return k(s, t, src, dst)


def _softmax_body(x_ref, o_ref):
    x = x_ref[...]
    m = jnp.max(x)
    e = jnp.exp(x - m)
    o_ref[...] = e / jnp.sum(e)


def _softmax(logits2d):
    return pl.pallas_call(
        _softmax_body,
        out_shape=jax.ShapeDtypeStruct(logits2d.shape, jnp.float32),
    )(logits2d)


def _scale_body(emb_hbm, src_hbm, dst_hbm, w_hbm, out_hbm,
                si, di, wq, rs, rd, sh_emb, gsem, osem, isem):
    wid = lax.axis_index("s") * NC + lax.axis_index("c")
    base = wid * EPW
    @pl.when(lax.axis_index("s") == 0)
    def _():
        pltpu.sync_copy(emb_hbm, sh_emb)

    plsc.subcore_barrier()

    def i_copies(k, s):
        return (
            pltpu.make_async_copy(
                src_hbm.at[pl.ds(base + k * B, B)], si.at[s], isem.at[s]),
            pltpu.make_async_copy(
                dst_hbm.at[pl.ds(base + k * B, B)], di.at[s], isem.at[s]),
            pltpu.make_async_copy(
                w_hbm.at[pl.ds(base + k * B, B)], wq.at[s], isem.at[s]),
        )

    def g_copies(k, s):
        return (
            pltpu.make_async_copy(
                sh_emb.at[si.at[s]], rs.at[s], gsem.at[s]),
            pltpu.make_async_copy(
                sh_emb.at[di.at[s]], rd.at[s], gsem.at[s]),
        )

    def o_copies(k, s):
        return (
            pltpu.make_async_copy(
                rs.at[s],
                out_hbm.at[pl.ds(base + k * B, B), pl.ds(0, D)],
                osem.at[s]),
            pltpu.make_async_copy(
                rd.at[s],
                out_hbm.at[pl.ds(base + k * B, B), pl.ds(D, D)],
                osem.at[s]),
        )

    def start(copies):
        for c in copies:
            c.start()

    def wait(copies):
        for c in copies:
            c.wait()

    def compute(s):
        @plsc.parallel_loop(0, B, step=1, unroll=4)
        def _(e):
            wb = plsc.load_gather(wq.at[s], [jnp.full((L,), e, jnp.int32)])
            for f in range(D // L):
                rs[s, e, pl.ds(f * L, L)] = rs[s, e, pl.ds(f * L, L)] * wb
                rd[s, e, pl.ds(f * L, L)] = rd[s, e, pl.ds(f * L, L)] * wb

    start(i_copies(0, 0))
    start(i_copies(1, 1))
    wait(i_copies(0, 0))
    start(g_copies(0, 0))

    def body(k, carry):
        s = lax.rem(k, 2)
        @pl.when(k + 1 < NB)
        def _():
            wait(i_copies(k + 1, 1 - s))
            start(g_copies(k + 1, 1 - s))

        wait(g_copies(k, s))
        compute(s)

        @pl.when(k + 2 < NB)
        def _():
            start(i_copies(k + 2, s))

        start(o_copies(k, s))
        wait(o_copies(k, s))
        return carry

    lax.fori_loop(0, NB, body, 0)


def _gather_scale(emb, src, dst, w):
    k = functools.partial(
        pl.kernel,
        out_type=jax.ShapeDtypeStruct((N_EDGES, 2 * D), jnp.float32),
        mesh=plsc.VectorSubcoreMesh(**_MESH),
        compiler_params=pltpu.CompilerParams(needs_layout_passes=False),
        scratch_types=[
            pltpu.VMEM((2, B), jnp.int32),
            pltpu.VMEM((2, B), jnp.int32),
            pltpu.VMEM((2, B), jnp.float32),
            pltpu.VMEM((2, B, D), jnp.float32),
            pltpu.VMEM((2, B, D), jnp.float32),
            pltpu.VMEM_SHARED((N_NODES, D), jnp.float32),
            pltpu.SemaphoreType.DMA((2,)),
            pltpu.SemaphoreType.DMA((2,)),
            pltpu.SemaphoreType.DMA((2,)),
        ],
    )(_scale_body)
    return k(emb, src, dst, w)


def kernel(node_embeddings, edge_index, attn_w, attn_b):
    emb = node_embeddings.astype(jnp.float32)
    src = edge_index[0].astype(jnp.int32)
    dst = edge_index[1].astype(jnp.int32)
    w2 = attn_w.reshape(2, D)
    st = _node_scores(emb, w2)
    logits = _edge_logits(st[0], st[1], src, dst)
    w = _softmax(logits.reshape(N_EDGES // D, D)).reshape(N_EDGES)
    return _gather_scale(emb, src, dst, w)


# speedup vs baseline: 1.3827x; 1.1930x over previous
import functools

import jax
import jax.numpy as jnp
from jax import lax
from jax.experimental import pallas as pl
from jax.experimental.pallas import tpu as pltpu
import jax.experimental.pallas.tpu_sc as plsc

N_NODES = 10000
N_EDGES = 320000
D = 128
L = 16
NC, NS = 2, 16
NW = NC * NS
EPW = N_EDGES // NW
B = 40
NB = EPW // B
S = 4

_MESH = dict(core_axis_name="c", subcore_axis_name="s", num_cores=NC,
             num_subcores=NS)


def _scores_body(emb_ref, w2_ref, out_ref):
    out_ref[...] = lax.dot_general(
        w2_ref[...], emb_ref[...], (((1,), (1,)), ((), ())),
        preferred_element_type=jnp.float32)


def _node_scores(emb, w2):
    return pl.pallas_call(
        _scores_body,
        out_shape=jax.ShapeDtypeStruct((2, N_NODES), jnp.float32),
    )(emb, w2)


def _logits_body(s_hbm, t_hbm, src_hbm, dst_hbm, out_hbm,
                 s_v, t_v, src_v, dst_v, lg_v):
    wid = lax.axis_index("s") * NC + lax.axis_index("c")
    base = wid * EPW
    pltpu.sync_copy(s_hbm, s_v)
    pltpu.sync_copy(t_hbm, t_v)
    pltpu.sync_copy(src_hbm.at[pl.ds(base, EPW)], src_v)
    pltpu.sync_copy(dst_hbm.at[pl.ds(base, EPW)], dst_v)

    def body(i, carry):
        o = i * L
        is_ = src_v[pl.ds(o, L)]
        id_ = dst_v[pl.ds(o, L)]
        sv = plsc.load_gather(s_v, [is_])
        tv = plsc.load_gather(t_v, [id_])
        lg_v[pl.ds(o, L)] = sv + tv
        return carry

    lax.fori_loop(0, EPW // L, body, 0)
    pltpu.sync_copy(lg_v, out_hbm.at[pl.ds(base, EPW)])


def _edge_logits(s, t, src, dst):
    k = functools.partial(
        pl.kernel,
        out_type=jax.ShapeDtypeStruct((N_EDGES,), jnp.float32),
        mesh=plsc.VectorSubcoreMesh(**_MESH),
        compiler_params=pltpu.CompilerParams(needs_layout_passes=False),
        scratch_types=[
            pltpu.VMEM((N_NODES,), jnp.float32),
            pltpu.VMEM((N_NODES,), jnp.float32),
            pltpu.VMEM((EPW,), jnp.int32),
            pltpu.VMEM((EPW,), jnp.int32),
            pltpu.VMEM((EPW,), jnp.float32),
        ],
    )(_logits_body)
    return k(s, t, src, dst)


def _softmax_body(x_ref, o_ref):
    x = x_ref[...]
    m = jnp.max(x)
    e = jnp.exp(x - m)
    o_ref[...] = e / jnp.sum(e)


def _softmax(logits2d):
    return pl.pallas_call(
        _softmax_body,
        out_shape=jax.ShapeDtypeStruct(logits2d.shape, jnp.float32),
    )(logits2d)


def _scale_body(emb_hbm, src_hbm, dst_hbm, w_hbm, out_hbm,
                si, di, wq, rs, rd, sh_emb, gsem, osem, isem):
    wid = lax.axis_index("s") * NC + lax.axis_index("c")
    base = wid * EPW
    @pl.when(lax.axis_index("s") == 0)
    def _():
        pltpu.sync_copy(emb_hbm, sh_emb)

    plsc.subcore_barrier()

    def i_copies(k, s):
        return (
            pltpu.make_async_copy(
                src_hbm.at[pl.ds(base + k * B, B)], si.at[s], isem.at[s]),
            pltpu.make_async_copy(
                dst_hbm.at[pl.ds(base + k * B, B)], di.at[s], isem.at[s]),
            pltpu.make_async_copy(
                w_hbm.at[pl.ds(base + k * B, B)], wq.at[s], isem.at[s]),
        )

    def g_copies(k, s):
        return (
            pltpu.make_async_copy(
                sh_emb.at[si.at[s]], rs.at[s], gsem.at[s]),
            pltpu.make_async_copy(
                sh_emb.at[di.at[s]], rd.at[s], gsem.at[s]),
        )

    def o_copies(k, s):
        return (
            pltpu.make_async_copy(
                rs.at[s],
                out_hbm.at[pl.ds(base + k * B, B), pl.ds(0, D)],
                osem.at[s]),
            pltpu.make_async_copy(
                rd.at[s],
                out_hbm.at[pl.ds(base + k * B, B), pl.ds(D, D)],
                osem.at[s]),
        )

    def start(copies):
        for c in copies:
            c.start()

    def wait(copies):
        for c in copies:
            c.wait()

    def compute(s):
        @plsc.parallel_loop(0, B, step=1, unroll=4)
        def _(e):
            wb = plsc.load_gather(wq.at[s], [jnp.full((L,), e, jnp.int32)])
            for f in range(D // L):
                rs[s, e, pl.ds(f * L, L)] = rs[s, e, pl.ds(f * L, L)] * wb
                rd[s, e, pl.ds(f * L, L)] = rd[s, e, pl.ds(f * L, L)] * wb

    for j in range(S):
        start(i_copies(j, j))
    wait(i_copies(0, 0))
    start(g_copies(0, 0))
    wait(i_copies(1, 1))
    start(g_copies(1, 1))

    def body(k, carry):
        s = lax.rem(k, S)
        s2 = lax.rem(k + 2, S)

        @pl.when(k >= 2)
        def _():
            wait(o_copies(k - 2, s2))

        @pl.when(k + 2 < NB)
        def _():
            wait(i_copies(k + 2, s2))
            start(g_copies(k + 2, s2))

        wait(g_copies(k, s))
        compute(s)

        @pl.when(k + S < NB)
        def _():
            start(i_copies(k + S, s))

        start(o_copies(k, s))
        return carry

    lax.fori_loop(0, NB, body, 0)
    wait(o_copies(NB - 2, (NB - 2) % S))
    wait(o_copies(NB - 1, (NB - 1) % S))


def _gather_scale(emb, src, dst, w):
    k = functools.partial(
        pl.kernel,
        out_type=jax.ShapeDtypeStruct((N_EDGES, 2 * D), jnp.float32),
        mesh=plsc.VectorSubcoreMesh(**_MESH),
        compiler_params=pltpu.CompilerParams(needs_layout_passes=False),
        scratch_types=[
            pltpu.VMEM((S, B), jnp.int32),
            pltpu.VMEM((S, B), jnp.int32),
            pltpu.VMEM((S, B), jnp.float32),
            pltpu.VMEM((S, B, D), jnp.float32),
            pltpu.VMEM((S, B, D), jnp.float32),
            pltpu.VMEM_SHARED((N_NODES, D), jnp.float32),
            pltpu.SemaphoreType.DMA((S,)),
            pltpu.SemaphoreType.DMA((S,)),
            pltpu.SemaphoreType.DMA((S,)),
        ],
    )(_scale_body)
    return k(emb, src, dst, w)


def kernel(node_embeddings, edge_index, attn_w, attn_b):
    emb = node_embeddings.astype(jnp.float32)
    src = edge_index[0].astype(jnp.int32)
    dst = edge_index[1].astype(jnp.int32)
    w2 = attn_w.reshape(2, D)
    st = _node_scores(emb, w2)
    logits = _edge_logits(st[0], st[1], src, dst)
    w = _softmax(logits.reshape(N_EDGES // D, D)).reshape(N_EDGES)
    return _gather_scale(emb, src, dst, w)


# speedup vs baseline: 1.3965x; 1.0100x over previous
import functools

import jax
import jax.numpy as jnp
from jax import lax
from jax.experimental import pallas as pl
from jax.experimental.pallas import tpu as pltpu
import jax.experimental.pallas.tpu_sc as plsc

N_NODES = 10000
N_EDGES = 320000
D = 128
L = 16
NC, NS = 2, 16
NW = NC * NS
EPW = N_EDGES // NW
B = 40
NB = EPW // B
S = 4

_MESH = dict(core_axis_name="c", subcore_axis_name="s", num_cores=NC,
             num_subcores=NS)


def _scores_body(emb_ref, w2_ref, out_ref):
    out_ref[...] = lax.dot_general(
        w2_ref[...], emb_ref[...], (((1,), (1,)), ((), ())),
        preferred_element_type=jnp.float32)


def _node_scores(emb, w2):
    return pl.pallas_call(
        _scores_body,
        out_shape=jax.ShapeDtypeStruct((2, N_NODES), jnp.float32),
    )(emb, w2)


def _logits_body(s_hbm, t_hbm, src_hbm, dst_hbm, out_hbm,
                 s_v, t_v, src_v, dst_v, lg_v):
    wid = lax.axis_index("s") * NC + lax.axis_index("c")
    base = wid * EPW
    pltpu.sync_copy(s_hbm, s_v)
    pltpu.sync_copy(t_hbm, t_v)
    pltpu.sync_copy(src_hbm.at[pl.ds(base, EPW)], src_v)
    pltpu.sync_copy(dst_hbm.at[pl.ds(base, EPW)], dst_v)

    @plsc.parallel_loop(0, EPW // L, step=1, unroll=4)
    def _(i):
        o = i * L
        is_ = src_v[pl.ds(o, L)]
        id_ = dst_v[pl.ds(o, L)]
        sv = plsc.load_gather(s_v, [is_])
        tv = plsc.load_gather(t_v, [id_])
        lg_v[pl.ds(o, L)] = sv + tv
    pltpu.sync_copy(lg_v, out_hbm.at[pl.ds(base, EPW)])


def _edge_logits(s, t, src, dst):
    k = functools.partial(
        pl.kernel,
        out_type=jax.ShapeDtypeStruct((N_EDGES,), jnp.float32),
        mesh=plsc.VectorSubcoreMesh(**_MESH),
        compiler_params=pltpu.CompilerParams(needs_layout_passes=False),
        scratch_types=[
            pltpu.VMEM((N_NODES,), jnp.float32),
            pltpu.VMEM((N_NODES,), jnp.float32),
            pltpu.VMEM((EPW,), jnp.int32),
            pltpu.VMEM((EPW,), jnp.int32),
            pltpu.VMEM((EPW,), jnp.float32),
        ],
    )(_logits_body)
    return k(s, t, src, dst)


def _softmax_body(x_ref, o_ref):
    x = x_ref[...]
    m = jnp.max(x)
    e = jnp.exp(x - m)
    o_ref[...] = e / jnp.sum(e)


def _softmax(logits2d):
    return pl.pallas_call(
        _softmax_body,
        out_shape=jax.ShapeDtypeStruct(logits2d.shape, jnp.float32),
    )(logits2d)


def _scale_body(emb_hbm, src_hbm, dst_hbm, w_hbm, out_hbm,
                si, di, wq, rs, rd, sh_emb, gsem, osem, isem):
    wid = lax.axis_index("s") * NC + lax.axis_index("c")
    base = wid * EPW
    @pl.when(lax.axis_index("s") == 0)
    def _():
        pltpu.sync_copy(emb_hbm, sh_emb)

    plsc.subcore_barrier()

    def i_copies(k, s):
        return (
            pltpu.make_async_copy(
                src_hbm.at[pl.ds(base + k * B, B)], si.at[s], isem.at[s]),
            pltpu.make_async_copy(
                dst_hbm.at[pl.ds(base + k * B, B)], di.at[s], isem.at[s]),
            pltpu.make_async_copy(
                w_hbm.at[pl.ds(base + k * B, B)], wq.at[s], isem.at[s]),
        )

    def g_copies(k, s):
        return (
            pltpu.make_async_copy(
                sh_emb.at[si.at[s]], rs.at[s], gsem.at[s]),
            pltpu.make_async_copy(
                sh_emb.at[di.at[s]], rd.at[s], gsem.at[s]),
        )

    def o_copies(k, s):
        return (
            pltpu.make_async_copy(
                rs.at[s],
                out_hbm.at[pl.ds(base + k * B, B), pl.ds(0, D)],
                osem.at[s]),
            pltpu.make_async_copy(
                rd.at[s],
                out_hbm.at[pl.ds(base + k * B, B), pl.ds(D, D)],
                osem.at[s]),
        )

    def start(copies):
        for c in copies:
            c.start()

    def wait(copies):
        for c in copies:
            c.wait()

    def compute(s):
        @plsc.parallel_loop(0, B, step=1, unroll=4)
        def _(e):
            wb = plsc.load_gather(wq.at[s], [jnp.full((L,), e, jnp.int32)])
            for f in range(D // L):
                rs[s, e, pl.ds(f * L, L)] = rs[s, e, pl.ds(f * L, L)] * wb
                rd[s, e, pl.ds(f * L, L)] = rd[s, e, pl.ds(f * L, L)] * wb

    for j in range(S):
        start(i_copies(j, j))
    wait(i_copies(0, 0))
    start(g_copies(0, 0))
    wait(i_copies(1, 1))
    start(g_copies(1, 1))

    def body(k, carry):
        s = lax.rem(k, S)
        s2 = lax.rem(k + 2, S)

        @pl.when(k >= 2)
        def _():
            wait(o_copies(k - 2, s2))

        @pl.when(k + 2 < NB)
        def _():
            wait(i_copies(k + 2, s2))
            start(g_copies(k + 2, s2))

        wait(g_copies(k, s))
        compute(s)

        @pl.when(k + S < NB)
        def _():
            start(i_copies(k + S, s))

        start(o_copies(k, s))
        return carry

    lax.fori_loop(0, NB, body, 0)
    wait(o_copies(NB - 2, (NB - 2) % S))
    wait(o_copies(NB - 1, (NB - 1) % S))


def _gather_scale(emb, src, dst, w):
    k = functools.partial(
        pl.kernel,
        out_type=jax.ShapeDtypeStruct((N_EDGES, 2 * D), jnp.float32),
        mesh=plsc.VectorSubcoreMesh(**_MESH),
        compiler_params=pltpu.CompilerParams(needs_layout_passes=False),
        scratch_types=[
            pltpu.VMEM((S, B), jnp.int32),
            pltpu.VMEM((S, B), jnp.int32),
            pltpu.VMEM((S, B), jnp.float32),
            pltpu.VMEM((S, B, D), jnp.float32),
            pltpu.VMEM((S, B, D), jnp.float32),
            pltpu.VMEM_SHARED((N_NODES, D), jnp.float32),
            pltpu.SemaphoreType.DMA((S,)),
            pltpu.SemaphoreType.DMA((S,)),
            pltpu.SemaphoreType.DMA((S,)),
        ],
    )(_scale_body)
    return k(emb, src, dst, w)


def kernel(node_embeddings, edge_index, attn_w, attn_b):
    emb = node_embeddings.astype(jnp.float32)
    src = edge_index[0].astype(jnp.int32)
    dst = edge_index[1].astype(jnp.int32)
    w2 = attn_w.reshape(2, D)
    st = _node_scores(emb, w2)
    logits = _edge_logits(st[0], st[1], src, dst)
    w = _softmax(logits.reshape(N_EDGES // D, D)).reshape(N_EDGES)
    return _gather_scale(emb, src, dst, w)
